# baseline (speedup 1.0000x reference)
import functools

import jax
import jax.numpy as jnp
from jax import lax
from jax.experimental import pallas as pl
from jax.experimental.pallas import tpu as pltpu
from jax.experimental.pallas import tpu_sc as plsc

B = 16384
NDIM = 3
VOCAB = 100000
RANK = 32
K1 = 32
LAMBD = 0.01

_NC = 2
_NS = 16
_NW = _NC * _NS
_BPW = B // _NW
_CHUNK = 128
_NCHUNK = _BPW // _CHUNK


def _sc_gather_body(idx_hbm, t0, t1, t2, t3, t4, t5, out_hbm,
                    idx_v, rows_a, rows_b, sem_a, sem_b):
    tabs = (t0, t1, t2, t3, t4, t5)
    wid = lax.axis_index("s") * _NC + lax.axis_index("c")
    base = wid * _BPW
    pltpu.sync_copy(idx_hbm.at[:, wid], idx_v)
    bufs = (rows_a, rows_b)
    sems = (sem_a, sem_b)
    nk = 6 * _NCHUNK
    specs = [(tabs[t], t % NDIM, j, t * B + base + j * _CHUNK)
             for t in range(6) for j in range(_NCHUNK)]

    def _issue(k):
        tab, dd, j, _ = specs[k]
        return pltpu.async_copy(tab.at[idx_v.at[dd, j]],
                                bufs[k % 2], sems[k % 2])

    inflight = {0: _issue(0), 1: _issue(1)}
    for k in range(nk):
        inflight.pop(k).wait()
        pltpu.sync_copy(bufs[k % 2], out_hbm.at[pl.ds(specs[k][3], _CHUNK)])
        if k + 2 < nk:
            inflight[k + 2] = _issue(k + 2)


def _sc_gather(idx, tables):
    mesh = plsc.VectorSubcoreMesh(core_axis_name="c", subcore_axis_name="s")
    kfn = pl.kernel(
        _sc_gather_body,
        out_type=jax.ShapeDtypeStruct((6 * B, RANK), jnp.float32),
        mesh=mesh,
        scratch_types=[
            pltpu.VMEM((NDIM, _NCHUNK, _CHUNK), jnp.int32),
            pltpu.VMEM((_CHUNK, RANK), jnp.float32),
            pltpu.VMEM((_CHUNK, RANK), jnp.float32),
            pltpu.SemaphoreType.DMA,
            pltpu.SemaphoreType.DMA,
        ],
        compiler_params=pltpu.CompilerParams(use_tc_tiling_on_sc=False),
    )
    return kfn(idx, *tables)


def _tc_body(g_ref, eps_ref, y_ref, o_ref, fs_ref):
    t = pl.program_id(0)
    u = t % 4
    prod = None
    for d in range(NDIM):
        mT = g_ref[d, pl.ds(u * 8, 8), :]
        lT = g_ref[NDIM + d, pl.ds(u * 8, 8), :]
        sT = lT * lT
        e = eps_ref[d]
        f = mT[None, :, :] + e * sT[None, :, :]
        prod = f if prod is None else prod * f
    part = jnp.sum(prod, axis=1)

    @pl.when(u == 0)
    def _():
        fs_ref[...] = part

    @pl.when(u != 0)
    def _():
        fs_ref[...] += part

    @pl.when(t == 0)
    def _():
        klp = jnp.float32(0.0)
        for d in range(NDIM):
            mT = g_ref[d]
            sT = g_ref[NDIM + d] ** 2
            s2 = sT * sT
            klp += jnp.sum(1.0 + jnp.log(s2) - mT * mT - s2)
        o_ref[...] = jnp.full((1, 1), 0.5 * LAMBD * klp, jnp.float32)

    @pl.when(u == 3)
    def _():
        sq = (fs_ref[...] - y_ref[...]) ** 2
        o_ref[...] += jnp.full((1, 1), 0.5 * jnp.sum(sq) / K1, jnp.float32)


def _tc_compute(g, eps_r, ys):
    return pl.pallas_call(
        _tc_body,
        grid=(16,),
        in_specs=[
            pl.BlockSpec((6, RANK, B), lambda t: (0, 0, 0)),
            pl.BlockSpec((NDIM, 8, 8, B), lambda t: (0, 0, t, 0)),
            pl.BlockSpec((1, B), lambda t: (0, 0)),
        ],
        out_specs=pl.BlockSpec((1, 1), lambda t: (0, 0)),
        out_shape=jax.ShapeDtypeStruct((1, 1), jnp.float32),
        scratch_shapes=[pltpu.VMEM((8, B), jnp.float32)],
    )(g, eps_r, ys)


def kernel(entries, ys, means_0, means_1, means_2,
           chols_0, chols_1, chols_2, eps):
    idx = entries.T.astype(jnp.int32).reshape(NDIM, _NW, _NCHUNK, _CHUNK)
    tables = (means_0, means_1, means_2, chols_0, chols_1, chols_2)
    g = jnp.swapaxes(_sc_gather(idx, tables).reshape(6, B, RANK), 1, 2)
    eps_t = eps.reshape(NDIM, B, 8, 128).transpose(0, 2, 3, 1)
    out = _tc_compute(g, eps_t, ys.reshape(1, B))
    return out.reshape(1)


# speedup vs baseline: 1.0669x; 1.0669x over previous
import functools

import jax
import jax.numpy as jnp
from jax import lax
from jax.experimental import pallas as pl
from jax.experimental.pallas import tpu as pltpu
from jax.experimental.pallas import tpu_sc as plsc

B = 16384
NDIM = 3
VOCAB = 100000
RANK = 32
K1 = 32
LAMBD = 0.01

_NC = 2
_NS = 16
_NW = _NC * _NS
_BPW = B // _NW
_CHUNK = 128
_NCHUNK = _BPW // _CHUNK


def _sc_gather_body(idx_hbm, t0, t1, t2, t3, t4, t5, out_hbm,
                    idx_v, rows_a, rows_b, sem_a, sem_b):
    tabs = (t0, t1, t2, t3, t4, t5)
    wid = lax.axis_index("s") * _NC + lax.axis_index("c")
    base = wid * _BPW
    pltpu.sync_copy(idx_hbm.at[:, wid], idx_v)
    bufs = (rows_a, rows_b)
    sems = (sem_a, sem_b)
    nk = 6 * _NCHUNK
    specs = [(tabs[t], t % NDIM, j, t * B + base + j * _CHUNK)
             for t in range(6) for j in range(_NCHUNK)]

    def _issue(k):
        tab, dd, j, _ = specs[k]
        return pltpu.async_copy(tab.at[idx_v.at[dd, j]],
                                bufs[k % 2], sems[k % 2])

    inflight = {0: _issue(0), 1: _issue(1)}
    for k in range(nk):
        inflight.pop(k).wait()
        pltpu.sync_copy(bufs[k % 2], out_hbm.at[pl.ds(specs[k][3], _CHUNK)])
        if k + 2 < nk:
            inflight[k + 2] = _issue(k + 2)


def _sc_gather(idx, tables):
    mesh = plsc.VectorSubcoreMesh(core_axis_name="c", subcore_axis_name="s")
    kfn = pl.kernel(
        _sc_gather_body,
        out_type=jax.ShapeDtypeStruct((6 * B, RANK), jnp.float32),
        mesh=mesh,
        scratch_types=[
            pltpu.VMEM((NDIM, _NCHUNK, _CHUNK), jnp.int32),
            pltpu.VMEM((_CHUNK, RANK), jnp.float32),
            pltpu.VMEM((_CHUNK, RANK), jnp.float32),
            pltpu.SemaphoreType.DMA,
            pltpu.SemaphoreType.DMA,
        ],
        compiler_params=pltpu.CompilerParams(use_tc_tiling_on_sc=False),
    )
    return kfn(idx, *tables)


_BS = 1024
_GRID = B // _BS


def _tc_body(g_ref, eps_ref, y_ref, o_ref):
    i = pl.program_id(0)
    prod = None
    klp = jnp.float32(0.0)
    for d in range(NDIM):
        md = g_ref[d]
        ld = g_ref[NDIM + d]
        s = ld * ld
        mT = jnp.swapaxes(md, 0, 1)
        sT = jnp.swapaxes(s, 0, 1)
        m_t = jnp.concatenate([mT, mT, mT, mT], axis=0)
        s_t = jnp.concatenate([sT, sT, sT, sT], axis=0)
        e = eps_ref[d]
        f = m_t[None, :, :] + e * s_t[None, :, :]
        prod = f if prod is None else prod * f
        s2 = s * s
        klp += jnp.sum(1.0 + jnp.log(s2) - md * md - s2)
    gr = lax.broadcasted_iota(jnp.int32, (4, 128), 0)
    gc = lax.broadcasted_iota(jnp.int32, (4, 128), 1) // RANK
    gsum = (gr == gc).astype(jnp.bfloat16)
    pb = prod.astype(jnp.bfloat16)
    fs = jnp.concatenate(
        [lax.dot_general(gsum, pb[si], (((1,), (0,)), ((), ())),
                         preferred_element_type=jnp.float32)
         for si in range(8)], axis=0)
    sq = (fs - y_ref[...]) ** 2
    contrib = 0.5 * (jnp.sum(sq) / K1 + LAMBD * klp)

    @pl.when(i == 0)
    def _():
        o_ref[...] = jnp.zeros((1, 1), jnp.float32)

    o_ref[...] += jnp.full((1, 1), contrib, jnp.float32)


def _tc_compute(g, eps_r, ys):
    return pl.pallas_call(
        _tc_body,
        grid=(_GRID,),
        in_specs=[
            pl.BlockSpec((6, _BS, RANK), lambda i: (0, i, 0)),
            pl.BlockSpec((NDIM, 8, 128, _BS), lambda i: (0, 0, 0, i)),
            pl.BlockSpec((1, _BS), lambda i: (0, i)),
        ],
        out_specs=pl.BlockSpec((1, 1), lambda i: (0, 0)),
        out_shape=jax.ShapeDtypeStruct((1, 1), jnp.float32),
    )(g, eps_r, ys)


def kernel(entries, ys, means_0, means_1, means_2,
           chols_0, chols_1, chols_2, eps):
    idx = entries.T.astype(jnp.int32).reshape(NDIM, _NW, _NCHUNK, _CHUNK)
    tables = (means_0, means_1, means_2, chols_0, chols_1, chols_2)
    g = _sc_gather(idx, tables).reshape(6, B, RANK)
    eps_t = eps.reshape(NDIM, B, 8, 128).transpose(0, 2, 3, 1)
    out = _tc_compute(g, eps_t, ys.reshape(1, B))
    return out.reshape(1)
